# Initial kernel scaffold; baseline (speedup 1.0000x reference)
import jax
import jax.numpy as jnp
from jax.experimental import pallas as pl


def kernel(x, gamma, beta, W, b):
    raise NotImplementedError("write your pallas kernel here")


# speedup vs baseline: 1.1096x; 1.1096x over previous
import functools

import jax
import jax.numpy as jnp
from jax.experimental import pallas as pl
from jax.experimental.pallas import tpu as pltpu

BN_EPS = 1e-5
LEAKY_SLOPE = 0.01

KB1 = 512
BM = 2048
BN = 512


def _bn_kernel(x_ref, gamma_ref, beta_ref, h_ref):
    x = x_ref[...]
    n = x.shape[0]
    mean = jnp.sum(x, axis=0, keepdims=True) * (1.0 / n)
    ex2 = jnp.sum(x * x, axis=0, keepdims=True) * (1.0 / n)
    var = ex2 - mean * mean
    s = gamma_ref[...] * jax.lax.rsqrt(var + BN_EPS)
    t = beta_ref[...] - mean * s
    h_ref[...] = (x * s + t).astype(jnp.bfloat16)


def _mm_kernel(h_ref, w_ref, b_ref, o_ref):
    acc = jax.lax.dot_general(
        h_ref[...], w_ref[...],
        dimension_numbers=(((1,), (1,)), ((), ())),
        preferred_element_type=jnp.float32,
    )
    y = acc + b_ref[...]
    o_ref[...] = jnp.where(y >= 0.0, y, LEAKY_SLOPE * y)


@functools.partial(jax.jit, donate_argnums=())
def kernel(x, gamma, beta, W, b):
    B, IN = x.shape
    OUT = W.shape[0]

    gamma2 = gamma.reshape(1, IN)
    beta2 = beta.reshape(1, IN)
    b2 = b.reshape(1, OUT)
    W16 = W.astype(jnp.bfloat16)

    h = pl.pallas_call(
        _bn_kernel,
        grid=(IN // KB1,),
        in_specs=[
            pl.BlockSpec((B, KB1), lambda k: (0, k)),
            pl.BlockSpec((1, KB1), lambda k: (0, k)),
            pl.BlockSpec((1, KB1), lambda k: (0, k)),
        ],
        out_specs=pl.BlockSpec((B, KB1), lambda k: (0, k)),
        out_shape=jax.ShapeDtypeStruct((B, IN), jnp.bfloat16),
        compiler_params=pltpu.CompilerParams(
            dimension_semantics=("parallel",),
        ),
    )(x, gamma2, beta2)

    out = pl.pallas_call(
        _mm_kernel,
        grid=(B // BM, OUT // BN),
        in_specs=[
            pl.BlockSpec((BM, IN), lambda m, n: (m, 0)),
            pl.BlockSpec((BN, IN), lambda m, n: (n, 0)),
            pl.BlockSpec((1, BN), lambda m, n: (0, n)),
        ],
        out_specs=pl.BlockSpec((BM, BN), lambda m, n: (m, n)),
        out_shape=jax.ShapeDtypeStruct((B, OUT), jnp.float32),
        compiler_params=pltpu.CompilerParams(
            dimension_semantics=("parallel", "arbitrary"),
        ),
    )(h, W16, b2)
    return out
